# baseline (speedup 1.0000x reference)
import functools

import jax
import jax.numpy as jnp
from jax.experimental import pallas as pl

N_Q = 8
K = 1024
D = 128
ROWS = 12000
BLOCK_ROWS = 1200


def _rowsum128(x):
    acc = x[:, 0:8]
    for t in range(1, 16):
        acc = acc + x[:, 8 * t:8 * t + 8]
    a = acc[:, :4] + acc[:, 4:8]
    b = a[:, :2] + a[:, 2:4]
    return b[:, 0:1] + b[:, 1:2]


def _colsum128(x):
    acc = x[0:8]
    for t in range(1, 16):
        acc = acc + x[8 * t:8 * t + 8]
    a = acc[:4] + acc[4:8]
    b = a[:2] + a[2:4]
    return b[0:1] + b[1:2]


def _rvq_body(z_ref, cb_ref, cbt_ref, out_ref, codes_ref):
    flat = z_ref[...]
    residual = flat
    quantized = jnp.zeros_like(flat)
    col = jax.lax.broadcasted_iota(jnp.int32, (BLOCK_ROWS, K), 1)
    codes_cols = []
    for q in range(N_Q):
        cb = cb_ref[q]
        cbt = cbt_ref[q]
        r2 = _rowsum128(residual * residual)
        c2 = _colsum128(cbt * cbt)
        rc = jax.lax.dot_general(residual.astype(jnp.bfloat16),
                                 cb.astype(jnp.bfloat16),
                                 (((1,), (1,)), ((), ())),
                                 preferred_element_type=jnp.float32)
        dist = r2 - 2.0 * rc + c2
        m = jnp.min(dist, axis=1, keepdims=True)
        idx = jnp.min(jnp.where(dist == m, col, K), axis=1, keepdims=True)
        onehot = (col == idx).astype(jnp.float32)
        sel = jax.lax.dot_general(onehot, cb, (((1,), (0,)), ((), ())),
                                  preferred_element_type=jnp.float32,
                                  precision=jax.lax.Precision.HIGHEST)
        quantized = quantized + sel
        residual = residual - sel
        codes_cols.append(idx)
    out_ref[...] = flat + (quantized - flat)
    codes_ref[...] = jnp.concatenate(codes_cols, axis=1)


@functools.partial(jax.jit, static_argnames=())
def kernel(z, codebooks):
    B, T, Dd = z.shape
    flat = z.reshape(ROWS, Dd)
    cbt = jnp.swapaxes(codebooks, 1, 2)
    grid = (ROWS // BLOCK_ROWS,)
    out, codes = pl.pallas_call(
        _rvq_body,
        grid=grid,
        in_specs=[
            pl.BlockSpec((BLOCK_ROWS, D), lambda i: (i, 0)),
            pl.BlockSpec((N_Q, K, D), lambda i: (0, 0, 0)),
            pl.BlockSpec((N_Q, D, K), lambda i: (0, 0, 0)),
        ],
        out_specs=[
            pl.BlockSpec((BLOCK_ROWS, D), lambda i: (i, 0)),
            pl.BlockSpec((BLOCK_ROWS, N_Q), lambda i: (i, 0)),
        ],
        out_shape=[
            jax.ShapeDtypeStruct((ROWS, D), jnp.float32),
            jax.ShapeDtypeStruct((ROWS, N_Q), jnp.int32),
        ],
    )(flat, codebooks, cbt)
    out = out.reshape(B, T, Dd)
    codes = codes.T.reshape(N_Q, B, T)
    return out, codes
